# Initial kernel scaffold; baseline (speedup 1.0000x reference)
import jax
import jax.numpy as jnp
from jax.experimental import pallas as pl


def kernel(vertex_normals, faces):
    raise NotImplementedError("write your pallas kernel here")


# speedup vs baseline: 33.1213x; 33.1213x over previous
import jax
import jax.numpy as jnp
from jax import lax
from jax.experimental import pallas as pl
from jax.experimental.pallas import tpu as pltpu
from jax.experimental.pallas import tpu_sc as plsc

B = 4
V = 100000
F = 200000
F_PAD = 204800
C = 1600
L = 16
GROUPS = C // L
NC = 2
NS = 16
TASKS = 6
W = TASKS * F_PAD // NS
BLOCKS_PER_TILE = F_PAD // C // NS
VALID_BLOCKS = F // C


def _body(vn_hbm, faces_hbm, out_hbm, part_hbm,
          table_v, i0_v, i1_v, i2_v, ob_v, p0_v, p1_v, p2_v):
    cid = lax.axis_index("c")
    sid = lax.axis_index("s")

    start = sid * W
    end = start + W
    task_a = start // F_PAD
    boundary = (task_a + 1) * F_PAD
    end_a = jnp.minimum(end, boundary)
    n_chunks_a = (end_a - start) // C
    n_chunks_b = (end - end_a) // C
    task_b = jnp.minimum(task_a + 1, TASKS - 1)

    def run_segment(task, face0, n_chunks):
        tbl_off = pl.multiple_of((cid * TASKS + task) * V, V)
        pltpu.sync_copy(vn_hbm.at[pl.ds(tbl_off, V)], table_v)

        def chunk_body(k, _):
            f0 = pl.multiple_of(face0 + k * C, C)
            pltpu.sync_copy(faces_hbm.at[pl.ds(f0, C)], i0_v)
            pltpu.sync_copy(faces_hbm.at[pl.ds(f0 + F_PAD, C)], i1_v)
            pltpu.sync_copy(faces_hbm.at[pl.ds(f0 + 2 * F_PAD, C)], i2_v)

            def grp(g, _):
                o = pl.multiple_of(g * L, L)
                a0 = i0_v[pl.ds(o, L)]
                a1 = i1_v[pl.ds(o, L)]
                a2 = i2_v[pl.ds(o, L)]
                v0 = plsc.load_gather(table_v, [a0])
                v1 = plsc.load_gather(table_v, [a1])
                v2 = plsc.load_gather(table_v, [a2])
                s = v0 + v1 + v2
                ob_v[pl.ds(o, L)] = s * s - v0 * v0 - v1 * v1 - v2 * v2
                return 0

            lax.fori_loop(0, GROUPS, grp, 0, unroll=4)
            part_off = pl.multiple_of((cid * TASKS + task) * F_PAD + f0, C)
            pltpu.sync_copy(ob_v, part_hbm.at[pl.ds(part_off, C)])
            return 0

        lax.fori_loop(0, n_chunks, chunk_body, 0)

    run_segment(task_a, start - task_a * F_PAD, n_chunks_a)

    @pl.when(n_chunks_b > 0)
    def _():
        run_segment(task_b, 0, n_chunks_b)

    plsc.subcore_barrier()

    for j in range(BLOCKS_PER_TILE):
        blk = sid * BLOCKS_PER_TILE + j

        @pl.when(blk < VALID_BLOCKS)
        def _():
            off = pl.multiple_of(blk * C, C)
            for bat in range(2):
                base = pl.multiple_of(
                    (cid * TASKS + bat * 3) * F_PAD + off, C)
                pltpu.sync_copy(part_hbm.at[pl.ds(base, C)], p0_v)
                pltpu.sync_copy(part_hbm.at[pl.ds(base + F_PAD, C)], p1_v)
                pltpu.sync_copy(part_hbm.at[pl.ds(base + 2 * F_PAD, C)], p2_v)

                def g2(g, _):
                    o = pl.multiple_of(g * L, L)
                    acc = p0_v[pl.ds(o, L)] + p1_v[pl.ds(o, L)] + p2_v[pl.ds(o, L)]
                    ob_v[pl.ds(o, L)] = 3.0 - 0.5 * acc
                    return 0

                lax.fori_loop(0, GROUPS, g2, 0, unroll=4)
                out_off = pl.multiple_of((cid * 2 + bat) * F + off, C)
                pltpu.sync_copy(ob_v, out_hbm.at[pl.ds(out_off, C)])


@jax.jit
def kernel(vertex_normals, faces):
    faces = jnp.squeeze(faces)
    vn_flat = jnp.transpose(vertex_normals, (0, 2, 1)).reshape(B * 3 * V)
    faces_flat = jnp.pad(
        jnp.transpose(faces), ((0, 0), (0, F_PAD - F))).reshape(3 * F_PAD)

    mesh = plsc.VectorSubcoreMesh(
        core_axis_name="c", subcore_axis_name="s", num_cores=NC, num_subcores=NS
    )
    run = pl.kernel(
        _body,
        out_type=(
            jax.ShapeDtypeStruct((B * F,), jnp.float32),
            jax.ShapeDtypeStruct((B * 3 * F_PAD,), jnp.float32),
        ),
        mesh=mesh,
        compiler_params=pltpu.CompilerParams(needs_layout_passes=False),
        scratch_types=[
            pltpu.VMEM((V,), jnp.float32),
            pltpu.VMEM((C,), jnp.int32),
            pltpu.VMEM((C,), jnp.int32),
            pltpu.VMEM((C,), jnp.int32),
            pltpu.VMEM((C,), jnp.float32),
            pltpu.VMEM((C,), jnp.float32),
            pltpu.VMEM((C,), jnp.float32),
            pltpu.VMEM((C,), jnp.float32),
        ],
    )
    out, _ = run(vn_flat, faces_flat)
    return out.reshape(B, F)
